# baseline (speedup 1.0000x reference)
import jax
import jax.numpy as jnp
from jax import lax
from jax.experimental import pallas as pl
from jax.experimental.pallas import tpu as pltpu

_V = 100000
_D = 16
_A = 1024
_B = 1024

_PK = 128 // _D
_VP = _V // _PK
_VTP = 2048
_VT = _VTP * _PK
_T = (_VP + _VTP - 1) // _VTP
_CH = 16
_U = _A

_HI = lax.Precision.DEFAULT
_STD = (((1,), (0,)), ((), ()))
_RT = (((1,), (1,)), ((), ()))


def _body(idx_ref, adap_ref, emb_ref, out_ref,
          au_scr, m_scr, acc_scr, rb_scr, n_scr):
    t = pl.program_id(0)

    @pl.when(t == 0)
    def _init():
        eidx = jnp.maximum(idx_ref[...] - _V, 0)
        ap = lax.broadcasted_iota(jnp.int32, (_B, _A), 1)
        aiota = _PK * (ap % (_A // _PK)) + ap // (_A // _PK)
        oh_ba = (eidx == aiota).astype(jnp.float32)
        ones_r = jnp.ones((1, _B), jnp.float32)
        counts = lax.dot_general(ones_r, oh_ba, _STD, precision=_HI)
        pres = (counts > 0.0).astype(jnp.float32)
        i_col = lax.broadcasted_iota(jnp.int32, (_A, _A), 0)
        j_row = lax.broadcasted_iota(jnp.int32, (_A, _A), 1)
        gt = (i_col < j_row).astype(jnp.float32)
        lt = (j_row < i_col).astype(jnp.float32)
        rank_r = lax.dot_general(pres, gt, _STD, precision=_HI)
        rank_c = lax.dot_general(lt, pres, _RT, precision=_HI)
        riota = lax.broadcasted_iota(jnp.int32, (_U, _A), 0)
        sel = ((riota == rank_r.astype(jnp.int32)) &
               (pres > 0.0)).astype(jnp.float32)
        adap_p = adap_ref[...]
        adap = jnp.concatenate(
            [adap_p[:, j * _D:(j + 1) * _D] for j in range(_PK)], axis=0)
        au_scr[...] = lax.dot_general(sel, adap, _STD,
                                      precision=_HI)
        rb_scr[...] = lax.dot_general(oh_ba, rank_c, _STD,
                                      precision=_HI)
        n_scr[0] = jnp.sum(pres).astype(jnp.int32)
        m_scr[...] = jnp.full((_U, 1), -jnp.inf, jnp.float32)
        acc_scr[...] = jnp.zeros((_U, _D), jnp.float32)

    is_last = t == _T - 1
    emb_p = lax.cond(
        is_last,
        lambda: jnp.where(
            lax.broadcasted_iota(jnp.int32, (_VTP, 1), 0) < _VP - (_T - 1) * _VTP,
            emb_ref[...], 0.0),
        lambda: emb_ref[...])
    emb = jnp.concatenate(
        [emb_p[:, j * _D:(j + 1) * _D] for j in range(_PK)], axis=0)
    n_ch = (n_scr[0] + _CH - 1) // _CH

    def _chunk(c, carry):
        rows = pl.ds(c * _CH, _CH)
        a_c = au_scr[rows, :]
        rel = lax.dot_general(a_c, emb, _RT,
                              preferred_element_type=jnp.float32)
        rel = lax.cond(
            is_last,
            lambda r: jnp.where(
                (lax.broadcasted_iota(jnp.int32, (1, _VT), 1) % _VTP)
                < _VP - (_T - 1) * _VTP,
                r, -jnp.inf),
            lambda r: r, rel)
        tmax = jnp.max(rel, axis=1, keepdims=True)
        m_old = m_scr[rows, :]
        m_new = jnp.maximum(m_old, tmax)
        mask = (rel == m_new).astype(jnp.float32)
        contrib = lax.dot_general(mask, emb, _STD,
                                  preferred_element_type=jnp.float32)
        acc_scr[rows, :] = jnp.where(tmax > m_old,
                                     contrib, acc_scr[rows, :] + contrib)
        m_scr[rows, :] = m_new
        return carry

    lax.fori_loop(0, n_ch, _chunk, 0)

    @pl.when(t == _T - 1)
    def _fin():
        uio = lax.broadcasted_iota(jnp.int32, (1, _U), 1)
        selb = (rb_scr[...].astype(jnp.int32) == uio).astype(jnp.float32)
        out_ref[...] = lax.dot_general(selb, acc_scr[...], _STD,
                                       precision=_HI)


def kernel(items_indices, constant_table, adaptive_table):
    idx_p = items_indices.reshape(_B, 1)
    tbl_p = constant_table.reshape(_VP, 128)
    adap_p = adaptive_table.reshape(_A * _D // 128, 128)
    out_p = pl.pallas_call(
        _body,
        grid=(_T,),
        in_specs=[
            pl.BlockSpec((_B, 1), lambda t: (0, 0)),
            pl.BlockSpec((_A * _D // 128, 128), lambda t: (0, 0)),
            pl.BlockSpec((_VTP, 128), lambda t: (t, 0)),
        ],
        out_specs=pl.BlockSpec((_B, _D), lambda t: (0, 0)),
        out_shape=jax.ShapeDtypeStruct((_B, _D), jnp.float32),
        scratch_shapes=[pltpu.VMEM((_U, _D), jnp.float32),
                        pltpu.VMEM((_U, 1), jnp.float32),
                        pltpu.VMEM((_U, _D), jnp.float32),
                        pltpu.VMEM((_B, 1), jnp.float32),
                        pltpu.SMEM((1,), jnp.int32)],
    )(idx_p, adap_p, tbl_p)
    return out_p


# speedup vs baseline: 1.5318x; 1.5318x over previous
import jax
import jax.numpy as jnp
from jax import lax
from jax.experimental import pallas as pl
from jax.experimental.pallas import tpu as pltpu

_V = 100000
_D = 16
_A = 1024
_B = 1024

_VT = 16384
_T = (_V + _VT - 1) // _VT
_OVL = _T * _VT - _V
_CH = 16
_U = _A

_HI = lax.Precision.DEFAULT
_STD = (((1,), (0,)), ((), ()))
_RT = (((1,), (1,)), ((), ()))


def _tile_copy(tbl_hbm, buf, sems, bi):
    start = jnp.minimum(bi * _VT, _V - _VT)
    return pltpu.make_async_copy(
        tbl_hbm.at[pl.ds(start, _VT), :], buf.at[bi % 2], sems.at[bi % 2])


def _body(idx_ref, adap_ref, tbl_hbm, out_ref,
          au_scr, m_scr, acc_scr, rb_scr, n_scr, buf, sems):
    t = pl.program_id(0)

    @pl.when(t == 0)
    def _prime():
        _tile_copy(tbl_hbm, buf, sems, 0).start()
        _tile_copy(tbl_hbm, buf, sems, 1).start()

    @pl.when((t > 0) & (t + 1 < _T))
    def _next():
        _tile_copy(tbl_hbm, buf, sems, t + 1).start()

    @pl.when(t == 0)
    def _init():
        eidx = jnp.maximum(idx_ref[...] - _V, 0)
        aiota = lax.broadcasted_iota(jnp.int32, (_B, _A), 1)
        oh_ba = (eidx == aiota).astype(jnp.float32)
        ones_r = jnp.ones((1, _B), jnp.float32)
        counts = lax.dot_general(ones_r, oh_ba, _STD, precision=_HI)
        pres = (counts > 0.0).astype(jnp.float32)
        i_col = lax.broadcasted_iota(jnp.int32, (_A, _A), 0)
        j_row = lax.broadcasted_iota(jnp.int32, (_A, _A), 1)
        gt = (i_col < j_row).astype(jnp.float32)
        lt = (j_row < i_col).astype(jnp.float32)
        rank_r = lax.dot_general(pres, gt, _STD, precision=_HI)
        rank_c = lax.dot_general(lt, pres, _RT, precision=_HI)
        riota = lax.broadcasted_iota(jnp.int32, (_U, _A), 0)
        sel = ((riota == rank_r.astype(jnp.int32)) &
               (pres > 0.0)).astype(jnp.float32)
        au_scr[...] = lax.dot_general(sel, adap_ref[...], _STD,
                                      precision=_HI)
        rb_scr[...] = lax.dot_general(oh_ba, rank_c, _STD,
                                      precision=_HI)
        n_scr[0] = jnp.sum(pres).astype(jnp.int32)
        m_scr[...] = jnp.full((_U, 1), -jnp.inf, jnp.float32)
        acc_scr[...] = jnp.zeros((_U, _D), jnp.float32)

    _tile_copy(tbl_hbm, buf, sems, t).wait()
    emb = buf[t % 2]
    is_last = t == _T - 1
    n_ch = (n_scr[0] + _CH - 1) // _CH

    def _chunk(c, carry):
        rows = pl.ds(c * _CH, _CH)
        a_c = au_scr[rows, :]
        rel = lax.dot_general(a_c, emb, _RT,
                              preferred_element_type=jnp.float32)
        rel = lax.cond(
            is_last,
            lambda r: jnp.where(
                lax.broadcasted_iota(jnp.int32, (1, _VT), 1) < _OVL,
                -jnp.inf, r),
            lambda r: r, rel)
        tmax = jnp.max(rel, axis=1, keepdims=True)
        m_old = m_scr[rows, :]
        m_new = jnp.maximum(m_old, tmax)
        mask = (rel == m_new).astype(jnp.float32)
        contrib = lax.dot_general(mask, emb, _STD,
                                  preferred_element_type=jnp.float32)
        acc_scr[rows, :] = jnp.where(tmax > m_old,
                                     contrib, acc_scr[rows, :] + contrib)
        m_scr[rows, :] = m_new
        return carry

    lax.fori_loop(0, n_ch, _chunk, 0)

    @pl.when(t == _T - 1)
    def _fin():
        uio = lax.broadcasted_iota(jnp.int32, (1, _U), 1)
        selb = (rb_scr[...].astype(jnp.int32) == uio).astype(jnp.float32)
        out_ref[...] = lax.dot_general(selb, acc_scr[...], _STD,
                                       precision=_HI)


def kernel(items_indices, constant_table, adaptive_table):
    idx_p = items_indices.reshape(_B, 1)
    return pl.pallas_call(
        _body,
        grid=(_T,),
        in_specs=[
            pl.BlockSpec((_B, 1), lambda t: (0, 0)),
            pl.BlockSpec((_A, _D), lambda t: (0, 0)),
            pl.BlockSpec(memory_space=pltpu.HBM),
        ],
        out_specs=pl.BlockSpec((_B, _D), lambda t: (0, 0)),
        out_shape=jax.ShapeDtypeStruct((_B, _D), jnp.float32),
        scratch_shapes=[pltpu.VMEM((_U, _D), jnp.float32),
                        pltpu.VMEM((_U, 1), jnp.float32),
                        pltpu.VMEM((_U, _D), jnp.float32),
                        pltpu.VMEM((_B, 1), jnp.float32),
                        pltpu.SMEM((1,), jnp.int32),
                        pltpu.VMEM((2, _VT, _D), jnp.float32),
                        pltpu.SemaphoreType.DMA((2,))],
    )(idx_p, adaptive_table, constant_table)
